# baseline (speedup 1.0000x reference)
import jax
import jax.numpy as jnp
from jax import lax
from jax.experimental import pallas as pl
from jax.experimental.pallas import tpu as pltpu
from jax.experimental.pallas import tpu_sc as plsc

L = 4
N = 100000
D = 128
B = 512

ROWS = L * N
CHUNK = 128
SC_N = 61440
SPAN = SC_N // 8
CNT = SPAN // CHUNK
NBUF = 5
LA = 4
ACC_ROWS = L * B

TC_BLK = 2000
TC_OFF = SC_N // TC_BLK
TC_BPL = N // TC_BLK - TC_OFF
BPL = N // TC_BLK


def _sc_body(hs_ref, bat_ref, out_ref, idx_dbuf, dbuf, stage, acc,
             sem_l0, sem_l1, sem_l2, sem_l3, sem_l4,
             sem_s0, sem_s1, sem_s2, sem_s3, sem_s4):
    c = lax.axis_index("c")
    s = lax.axis_index("s")
    loc_l = s // 8
    sl = s % 8
    layer = 2 * c + loc_l
    tile_base = layer * N + sl * SPAN

    sem_ld = (sem_l0, sem_l1, sem_l2, sem_l3, sem_l4)
    sem_sc = (sem_s0, sem_s1, sem_s2, sem_s3, sem_s4)
    dslot = tuple(dbuf.at[i] for i in range(NBUF))
    islot = tuple(idx_dbuf.at[i] for i in range(NBUF))

    def loads_start(j, slot):
        base = tile_base + j * CHUNK
        pltpu.async_copy(bat_ref.at[pl.ds(base, CHUNK)], islot[slot],
                         sem_ld[slot])
        pltpu.async_copy(hs_ref.at[pl.ds(base, CHUNK), :], dslot[slot],
                         sem_ld[slot])

    def loads_wait(slot):
        pltpu.make_async_copy(bat_ref.at[pl.ds(0, CHUNK)], islot[slot],
                              sem_ld[slot]).wait()
        pltpu.make_async_copy(hs_ref.at[pl.ds(0, CHUNK), :], dslot[slot],
                              sem_ld[slot]).wait()

    def fix_idx(slot):
        for g in range(8):
            v = idx_dbuf[slot, pl.ds(g * 16, 16)] + loc_l * B
            idx_dbuf[slot, pl.ds(g * 16, 16)] = v

    def scat_start(slot):
        pltpu.async_copy(dslot[slot], acc.at[islot[slot]],
                         sem_sc[slot], add=True)

    def scat_wait(slot):
        pltpu.make_async_copy(dslot[slot], acc.at[islot[slot]],
                              sem_sc[slot]).wait()

    for j in range(LA):
        loads_start(j, j)

    zeros16 = jnp.zeros((16,), jnp.float32)
    for r in range(32):
        for g in range(8):
            stage[r, pl.ds(g * 16, 16)] = zeros16
    for k in range(2):
        pltpu.sync_copy(stage, acc.at[pl.ds(s * 64 + k * 32, 32), :])

    plsc.subcore_barrier()

    def quint_step(t, carry):
        for k in range(5):
            j = 5 * t + k
            loads_wait(k)
            fix_idx(k)
            scat_start(k)

            @pl.when(j >= 5 - LA)
            def _ws():
                scat_wait((k + LA) % 5)

            @pl.when(j + LA < CNT)
            def _nl():
                loads_start(j + LA, (k + LA) % 5)

        return carry

    lax.fori_loop(0, CNT // 5, quint_step, 0)

    for j in range(CNT - (5 - LA), CNT):
        scat_wait(j % 5)

    plsc.subcore_barrier()

    for k in range(2):
        pltpu.sync_copy(acc.at[pl.ds(s * 64 + k * 32, 32), :], stage)
        pltpu.sync_copy(
            stage, out_ref.at[pl.ds(c * 1024 + s * 64 + k * 32, 32), :])


_sc_segsum = pl.kernel(
    _sc_body,
    out_type=jax.ShapeDtypeStruct((ACC_ROWS, D), jnp.float32),
    mesh=plsc.VectorSubcoreMesh(core_axis_name="c", subcore_axis_name="s"),
    scratch_types=[
        pltpu.VMEM((NBUF, CHUNK), jnp.int32),
        pltpu.VMEM((NBUF, CHUNK, D), jnp.float32),
        pltpu.VMEM((32, D), jnp.float32),
        pltpu.VMEM_SHARED((2 * B, D), jnp.float32),
    ] + [pltpu.SemaphoreType.DMA] * 10,
)


def _tc_seg_body(idx_ref, x_ref, o_ref):
    t = pl.program_id(1)

    @pl.when(t == 0)
    def _init():
        o_ref[...] = jnp.zeros((B, D), jnp.float32)

    seg = jax.lax.broadcasted_iota(jnp.int32, (B, TC_BLK), 0)
    rows = jax.lax.broadcasted_iota(jnp.int32, (B, TC_BLK), 1)
    thr = jnp.maximum(0, SC_N - (TC_OFF + t) * TC_BLK)
    oh = ((seg == idx_ref[...][0, 0]) & (rows >= thr)).astype(jnp.bfloat16)
    x = x_ref[...].astype(jnp.bfloat16)
    o_ref[...] += lax.dot_general(oh, x, (((1,), (0,)), ((), ())),
                                  preferred_element_type=jnp.float32)


def _tc_segsum(bat3, hs2):
    return pl.pallas_call(
        _tc_seg_body,
        grid=(L, TC_BPL),
        in_specs=[
            pl.BlockSpec((1, 1, TC_BLK), lambda l, t: (l * BPL + TC_OFF + t, 0, 0)),
            pl.BlockSpec((TC_BLK, D), lambda l, t: (l * BPL + TC_OFF + t, 0)),
        ],
        out_specs=pl.BlockSpec((B, D), lambda l, t: (l, 0)),
        out_shape=jax.ShapeDtypeStruct((ACC_ROWS, D), jnp.float32),
    )(bat3, hs2)


def _proj_body(xs_ref, xt_ref, w_ref, b_ref, o_ref):
    w = w_ref[...]
    r = jnp.broadcast_to(b_ref[...], (B, D))
    for l in range(L):
        x = xs_ref[pl.ds(l * B, B), :] + xt_ref[pl.ds(l * B, B), :]
        wl = w[:, l * D:(l + 1) * D]
        r = r + lax.dot_general(x, wl, (((1,), (1,)), ((), ())),
                                preferred_element_type=jnp.float32)
    o_ref[...] = r


def _project(parts_sc, parts_tc, W, b2):
    return pl.pallas_call(
        _proj_body,
        out_shape=jax.ShapeDtypeStruct((B, D), jnp.float32),
    )(parts_sc, parts_tc, W, b2)


@jax.jit
def kernel(hs, batches, W, b):
    hs2 = hs.reshape(ROWS, D)
    bat1 = batches.reshape(ROWS).astype(jnp.int32)
    bat3 = bat1.reshape(ROWS // TC_BLK, 1, TC_BLK)
    parts_sc = _sc_segsum(hs2, bat1)
    parts_tc = _tc_segsum(bat3, hs2)
    return _project(parts_sc, parts_tc, W, b.reshape(1, D))


# speedup vs baseline: 1.1114x; 1.1114x over previous
import jax
import jax.numpy as jnp
from jax import lax
from jax.experimental import pallas as pl
from jax.experimental.pallas import tpu as pltpu
from jax.experimental.pallas import tpu_sc as plsc

L = 4
N = 100000
D = 128
B = 512

ROWS = L * N
CHUNK = 128
SC_N = 66560
SPAN = SC_N // 8
CNT = SPAN // CHUNK
NBUF = 5
LA = 4
ACC_ROWS = L * B

TC_BLK = 2000
TC_OFF = SC_N // TC_BLK
TC_BPL = N // TC_BLK - TC_OFF
BPL = N // TC_BLK


def _sc_body(hs_ref, bat_ref, out_ref, idx_dbuf, dbuf, stage, acc,
             sem_l0, sem_l1, sem_l2, sem_l3, sem_l4,
             sem_s0, sem_s1, sem_s2, sem_s3, sem_s4):
    c = lax.axis_index("c")
    s = lax.axis_index("s")
    loc_l = s // 8
    sl = s % 8
    layer = 2 * c + loc_l
    tile_base = layer * N + sl * SPAN

    sem_ld = (sem_l0, sem_l1, sem_l2, sem_l3, sem_l4)
    sem_sc = (sem_s0, sem_s1, sem_s2, sem_s3, sem_s4)
    dslot = tuple(dbuf.at[i] for i in range(NBUF))
    islot = tuple(idx_dbuf.at[i] for i in range(NBUF))

    def loads_start(j, slot):
        base = tile_base + j * CHUNK
        pltpu.async_copy(bat_ref.at[pl.ds(base, CHUNK)], islot[slot],
                         sem_ld[slot])
        pltpu.async_copy(hs_ref.at[pl.ds(base, CHUNK), :], dslot[slot],
                         sem_ld[slot])

    def loads_wait(slot):
        pltpu.make_async_copy(bat_ref.at[pl.ds(0, CHUNK)], islot[slot],
                              sem_ld[slot]).wait()
        pltpu.make_async_copy(hs_ref.at[pl.ds(0, CHUNK), :], dslot[slot],
                              sem_ld[slot]).wait()

    def fix_idx(slot):
        for g in range(8):
            v = idx_dbuf[slot, pl.ds(g * 16, 16)] + loc_l * B
            idx_dbuf[slot, pl.ds(g * 16, 16)] = v

    def scat_start(slot):
        pltpu.async_copy(dslot[slot], acc.at[islot[slot]],
                         sem_sc[slot], add=True)

    def scat_wait(slot):
        pltpu.make_async_copy(dslot[slot], acc.at[islot[slot]],
                              sem_sc[slot]).wait()

    for j in range(LA):
        loads_start(j, j)

    zeros16 = jnp.zeros((16,), jnp.float32)
    for r in range(32):
        for g in range(8):
            stage[r, pl.ds(g * 16, 16)] = zeros16
    for k in range(2):
        pltpu.sync_copy(stage, acc.at[pl.ds(s * 64 + k * 32, 32), :])

    plsc.subcore_barrier()

    def quint_step(t, carry):
        for k in range(5):
            j = 5 * t + k
            loads_wait(k)
            fix_idx(k)
            scat_start(k)

            @pl.when(j >= 5 - LA)
            def _ws():
                scat_wait((k + LA) % 5)

            @pl.when(j + LA < CNT)
            def _nl():
                loads_start(j + LA, (k + LA) % 5)

        return carry

    lax.fori_loop(0, CNT // 5, quint_step, 0)

    for j in range(CNT - (5 - LA), CNT):
        scat_wait(j % 5)

    plsc.subcore_barrier()

    for k in range(2):
        pltpu.sync_copy(acc.at[pl.ds(s * 64 + k * 32, 32), :], stage)
        pltpu.sync_copy(
            stage, out_ref.at[pl.ds(c * 1024 + s * 64 + k * 32, 32), :])


_sc_segsum = pl.kernel(
    _sc_body,
    out_type=jax.ShapeDtypeStruct((ACC_ROWS, D), jnp.float32),
    mesh=plsc.VectorSubcoreMesh(core_axis_name="c", subcore_axis_name="s"),
    scratch_types=[
        pltpu.VMEM((NBUF, CHUNK), jnp.int32),
        pltpu.VMEM((NBUF, CHUNK, D), jnp.float32),
        pltpu.VMEM((32, D), jnp.float32),
        pltpu.VMEM_SHARED((2 * B, D), jnp.float32),
    ] + [pltpu.SemaphoreType.DMA] * 10,
)


def _tc_seg_body(idx_ref, x_ref, o_ref):
    t = pl.program_id(1)

    @pl.when(t == 0)
    def _init():
        o_ref[...] = jnp.zeros((B, D), jnp.float32)

    seg = jax.lax.broadcasted_iota(jnp.int32, (B, TC_BLK), 0)
    rows = jax.lax.broadcasted_iota(jnp.int32, (B, TC_BLK), 1)
    thr = jnp.maximum(0, SC_N - (TC_OFF + t) * TC_BLK)
    oh = ((seg == idx_ref[...][0, 0]) & (rows >= thr)).astype(jnp.bfloat16)
    x = x_ref[...].astype(jnp.bfloat16)
    o_ref[...] += lax.dot_general(oh, x, (((1,), (0,)), ((), ())),
                                  preferred_element_type=jnp.float32)


def _tc_segsum(bat3, hs2):
    return pl.pallas_call(
        _tc_seg_body,
        grid=(L, TC_BPL),
        in_specs=[
            pl.BlockSpec((1, 1, TC_BLK), lambda l, t: (l * BPL + TC_OFF + t, 0, 0)),
            pl.BlockSpec((TC_BLK, D), lambda l, t: (l * BPL + TC_OFF + t, 0)),
        ],
        out_specs=pl.BlockSpec((B, D), lambda l, t: (l, 0)),
        out_shape=jax.ShapeDtypeStruct((ACC_ROWS, D), jnp.float32),
    )(bat3, hs2)


def _proj_body(xs_ref, xt_ref, w_ref, b_ref, o_ref):
    w = w_ref[...]
    r = jnp.broadcast_to(b_ref[...], (B, D))
    for l in range(L):
        x = xs_ref[pl.ds(l * B, B), :] + xt_ref[pl.ds(l * B, B), :]
        wl = w[:, l * D:(l + 1) * D]
        r = r + lax.dot_general(x, wl, (((1,), (1,)), ((), ())),
                                preferred_element_type=jnp.float32)
    o_ref[...] = r


def _project(parts_sc, parts_tc, W, b2):
    return pl.pallas_call(
        _proj_body,
        out_shape=jax.ShapeDtypeStruct((B, D), jnp.float32),
    )(parts_sc, parts_tc, W, b2)


@jax.jit
def kernel(hs, batches, W, b):
    hs2 = hs.reshape(ROWS, D)
    bat1 = batches.reshape(ROWS).astype(jnp.int32)
    bat3 = bat1.reshape(ROWS // TC_BLK, 1, TC_BLK)
    parts_sc = _sc_segsum(hs2, bat1)
    parts_tc = _tc_segsum(bat3, hs2)
    return _project(parts_sc, parts_tc, W, b.reshape(1, D))
